# baseline (speedup 1.0000x reference)
import functools

import jax
import jax.numpy as jnp
from jax.experimental import pallas as pl

H = 1024
E = 8
I = 1024
IS = 2048
N_GROUP = 4
GSIZE = E // N_GROUP
TOPK_GROUP = 2
TOP_K = 2
ROUTED_SCALING = 2.5

T = 2048
TT = 256
NT = T // TT


def _router_body(xt_ref, wr_ref, bias_ref, comb_ref):
    logits = jax.lax.dot_general(
        wr_ref[...], xt_ref[...], (((1,), (0,)), ((), ())),
        preferred_element_type=jnp.float32)
    scores = jax.nn.sigmoid(logits)
    choice = scores + bias_ref[...]

    ch = [choice[e:e + 1, :] for e in range(E)]
    sc = [scores[e:e + 1, :] for e in range(E)]

    gs = [ch[2 * j] + ch[2 * j + 1] for j in range(N_GROUP)]
    gmask = []
    for j in range(N_GROUP):
        rank = jnp.zeros_like(gs[j])
        for k in range(N_GROUP):
            if k == j:
                continue
            if k < j:
                beats = (gs[k] >= gs[j])
            else:
                beats = (gs[k] > gs[j])
            rank = rank + beats.astype(jnp.float32)
        gmask.append(rank < float(TOPK_GROUP))

    ms = [jnp.where(gmask[e // GSIZE], ch[e], 0.0) for e in range(E)]

    sel = []
    for e in range(E):
        rank = jnp.zeros_like(ms[e])
        for f in range(E):
            if f == e:
                continue
            if f < e:
                beats = (ms[f] >= ms[e])
            else:
                beats = (ms[f] > ms[e])
            rank = rank + beats.astype(jnp.float32)
        sel.append(rank < float(TOP_K))

    w = [jnp.where(sel[e], sc[e], 0.0) for e in range(E)]
    denom = w[0]
    for e in range(1, E):
        denom = denom + w[e]
    denom = denom + 1e-20
    for e in range(E):
        comb_ref[e:e + 1, :] = (w[e] / denom) * ROUTED_SCALING


def _experts_body(xt_ref, gu_ref, d_ref, comb_ref, out_ref):
    e = pl.program_id(1)
    xb = xt_ref[...]
    gate = jax.lax.dot_general(
        gu_ref[0, :I, :], xb, (((1,), (0,)), ((), ())),
        preferred_element_type=jnp.float32)
    up = jax.lax.dot_general(
        gu_ref[0, I:, :], xb, (((1,), (0,)), ((), ())),
        preferred_element_type=jnp.float32)
    h = (gate * jax.nn.sigmoid(gate)) * up
    y = jax.lax.dot_general(
        d_ref[0], h, (((1,), (0,)), ((), ())),
        preferred_element_type=jnp.float32)
    contrib = y * comb_ref[0]

    @pl.when(e == 0)
    def _init():
        out_ref[...] = contrib

    @pl.when(e != 0)
    def _acc():
        out_ref[...] = out_ref[...] + contrib


def _shared_body(xt_ref, gw_ref, uw_ref, dw_ref, routed_ref, out_ref):
    xb = xt_ref[...]
    gate = jax.lax.dot_general(
        gw_ref[...], xb, (((1,), (0,)), ((), ())),
        preferred_element_type=jnp.float32)
    up = jax.lax.dot_general(
        uw_ref[...], xb, (((1,), (0,)), ((), ())),
        preferred_element_type=jnp.float32)
    h = (gate * jax.nn.sigmoid(gate)) * up
    y = jax.lax.dot_general(
        dw_ref[...], h, (((1,), (0,)), ((), ())),
        preferred_element_type=jnp.float32)
    out_ref[...] = y + routed_ref[...]


@jax.jit
def _run(x, router_weight, bias, gate_up_proj, down_proj,
         shared_gate_w, shared_up_w, shared_down_w):
    xt = x.T
    bias_col = bias.reshape(E, 1)

    comb = pl.pallas_call(
        _router_body,
        out_shape=jax.ShapeDtypeStruct((E, T), jnp.float32),
    )(xt, router_weight, bias_col)

    comb3 = comb.reshape(E, 1, T)

    routed_t = pl.pallas_call(
        _experts_body,
        grid=(NT, E),
        in_specs=[
            pl.BlockSpec((H, TT), lambda t, e: (0, t)),
            pl.BlockSpec((1, 2 * I, H), lambda t, e: (e, 0, 0)),
            pl.BlockSpec((1, H, I), lambda t, e: (e, 0, 0)),
            pl.BlockSpec((1, 1, TT), lambda t, e: (e, 0, t)),
        ],
        out_specs=pl.BlockSpec((H, TT), lambda t, e: (0, t)),
        out_shape=jax.ShapeDtypeStruct((H, T), jnp.float32),
    )(xt, gate_up_proj, down_proj, comb3)

    out_t = pl.pallas_call(
        _shared_body,
        grid=(NT,),
        in_specs=[
            pl.BlockSpec((H, TT), lambda t: (0, t)),
            pl.BlockSpec((IS, H), lambda t: (0, 0)),
            pl.BlockSpec((IS, H), lambda t: (0, 0)),
            pl.BlockSpec((H, IS), lambda t: (0, 0)),
            pl.BlockSpec((H, TT), lambda t: (0, t)),
        ],
        out_specs=pl.BlockSpec((H, TT), lambda t: (0, t)),
        out_shape=jax.ShapeDtypeStruct((H, T), jnp.float32),
    )(xt, shared_gate_w, shared_up_w, shared_down_w, routed_t)

    return out_t.T


def kernel(hidden_states, router_weight, e_score_correction_bias,
           gate_up_proj, down_proj, shared_gate_w, shared_up_w,
           shared_down_w):
    B, S, Hd = hidden_states.shape
    x = hidden_states.reshape(-1, Hd)
    out = _run(x, router_weight, e_score_correction_bias, gate_up_proj,
               down_proj, shared_gate_w, shared_up_w, shared_down_w)
    return out.reshape(B, S, Hd)


# speedup vs baseline: 1.6246x; 1.6246x over previous
import jax
import jax.numpy as jnp
from jax.experimental import pallas as pl

H = 1024
E = 8
I = 1024
IS = 2048
N_GROUP = 4
GSIZE = E // N_GROUP
TOPK_GROUP = 2
TOP_K = 2
ROUTED_SCALING = 2.5

T = 2048
IC = 512
NIC = I // IC
ISC = 512
NISC = IS // ISC

BF = jnp.bfloat16
F32 = jnp.float32


def _router_body(xt_ref, wr_ref, bias_ref, comb_ref):
    logits = jax.lax.dot_general(
        wr_ref[...], xt_ref[...], (((1,), (0,)), ((), ())),
        preferred_element_type=F32)
    scores = jax.nn.sigmoid(logits)
    choice = scores + bias_ref[...]

    ch = [choice[e:e + 1, :] for e in range(E)]
    sc = [scores[e:e + 1, :] for e in range(E)]

    gs = [ch[2 * j] + ch[2 * j + 1] for j in range(N_GROUP)]
    gmask = []
    for j in range(N_GROUP):
        rank = jnp.zeros_like(gs[j])
        for k in range(N_GROUP):
            if k == j:
                continue
            beats = (gs[k] >= gs[j]) if k < j else (gs[k] > gs[j])
            rank = rank + beats.astype(F32)
        gmask.append(rank < float(TOPK_GROUP))

    ms = [jnp.where(gmask[e // GSIZE], ch[e], 0.0) for e in range(E)]

    sel = []
    for e in range(E):
        rank = jnp.zeros_like(ms[e])
        for f in range(E):
            if f == e:
                continue
            beats = (ms[f] >= ms[e]) if f < e else (ms[f] > ms[e])
            rank = rank + beats.astype(F32)
        sel.append(rank < float(TOP_K))

    w = [jnp.where(sel[e], sc[e], 0.0) for e in range(E)]
    denom = w[0]
    for e in range(1, E):
        denom = denom + w[e]
    denom = denom + 1e-20
    for e in range(E):
        comb_ref[e:e + 1, :] = (w[e] / denom) * ROUTED_SCALING


def _experts_body(xt_ref, gu_ref, d_ref, comb_ref, out_ref):
    e = pl.program_id(0)
    xb = xt_ref[...]
    y = None
    for c in range(NIC):
        gw = gu_ref[0, c * IC:(c + 1) * IC, :].astype(BF)
        uw = gu_ref[0, I + c * IC:I + (c + 1) * IC, :].astype(BF)
        gate = jax.lax.dot_general(
            gw, xb, (((1,), (0,)), ((), ())), preferred_element_type=F32)
        up = jax.lax.dot_general(
            uw, xb, (((1,), (0,)), ((), ())), preferred_element_type=F32)
        h = ((gate * jax.nn.sigmoid(gate)) * up).astype(BF)
        dw = d_ref[0, :, c * IC:(c + 1) * IC].astype(BF)
        yc = jax.lax.dot_general(
            dw, h, (((1,), (0,)), ((), ())), preferred_element_type=F32)
        y = yc if y is None else y + yc
    contrib = y * comb_ref[0]

    @pl.when(e == 0)
    def _init():
        out_ref[...] = contrib

    @pl.when(e != 0)
    def _acc():
        out_ref[...] = out_ref[...] + contrib


def _shared_body(xt_ref, gw_ref, uw_ref, dw_ref, routed_ref, out_ref):
    c = pl.program_id(0)
    xb = xt_ref[...]
    gate = jax.lax.dot_general(
        gw_ref[...].astype(BF), xb, (((1,), (0,)), ((), ())),
        preferred_element_type=F32)
    up = jax.lax.dot_general(
        uw_ref[...].astype(BF), xb, (((1,), (0,)), ((), ())),
        preferred_element_type=F32)
    h = ((gate * jax.nn.sigmoid(gate)) * up).astype(BF)
    y = jax.lax.dot_general(
        dw_ref[...].astype(BF), h, (((1,), (0,)), ((), ())),
        preferred_element_type=F32)

    @pl.when(c == 0)
    def _init():
        out_ref[...] = y

    @pl.when((c != 0) & (c != NISC - 1))
    def _acc():
        out_ref[...] = out_ref[...] + y

    @pl.when((c == NISC - 1) & (c != 0))
    def _fin():
        out_ref[...] = out_ref[...] + y + routed_ref[...]


@jax.jit
def _run(x, router_weight, bias, gate_up_proj, down_proj,
         shared_gate_w, shared_up_w, shared_down_w):
    xt = x.T
    xt_bf = xt.astype(BF)
    bias_col = bias.reshape(E, 1)

    comb = pl.pallas_call(
        _router_body,
        out_shape=jax.ShapeDtypeStruct((E, T), F32),
    )(xt, router_weight, bias_col)

    comb3 = comb.reshape(E, 1, T)

    routed_t = pl.pallas_call(
        _experts_body,
        grid=(E,),
        in_specs=[
            pl.BlockSpec((H, T), lambda e: (0, 0)),
            pl.BlockSpec((1, 2 * I, H), lambda e: (e, 0, 0)),
            pl.BlockSpec((1, H, I), lambda e: (e, 0, 0)),
            pl.BlockSpec((1, 1, T), lambda e: (e, 0, 0)),
        ],
        out_specs=pl.BlockSpec((H, T), lambda e: (0, 0)),
        out_shape=jax.ShapeDtypeStruct((H, T), F32),
    )(xt_bf, gate_up_proj, down_proj, comb3)

    out_t = pl.pallas_call(
        _shared_body,
        grid=(NISC,),
        in_specs=[
            pl.BlockSpec((H, T), lambda c: (0, 0)),
            pl.BlockSpec((ISC, H), lambda c: (c, 0)),
            pl.BlockSpec((ISC, H), lambda c: (c, 0)),
            pl.BlockSpec((H, ISC), lambda c: (0, c)),
            pl.BlockSpec((H, T), lambda c: (0, 0)),
        ],
        out_specs=pl.BlockSpec((H, T), lambda c: (0, 0)),
        out_shape=jax.ShapeDtypeStruct((H, T), F32),
    )(xt_bf, shared_gate_w, shared_up_w, shared_down_w, routed_t)

    return out_t.T


def kernel(hidden_states, router_weight, e_score_correction_bias,
           gate_up_proj, down_proj, shared_gate_w, shared_up_w,
           shared_down_w):
    B, S, Hd = hidden_states.shape
    x = hidden_states.reshape(-1, Hd)
    out = _run(x, router_weight, e_score_correction_bias, gate_up_proj,
               down_proj, shared_gate_w, shared_up_w, shared_down_w)
    return out.reshape(B, S, Hd)
